# baseline (speedup 1.0000x reference)
import functools

import jax
import jax.numpy as jnp
from jax import lax
from jax.experimental import pallas as pl
from jax.experimental.pallas import tpu as pltpu
from jax.experimental.pallas import tpu_sc as plsc

_N = 10000
_E = 320000
_D = 128
_K = 5

_NC = 2
_NS = 16
_NW = _NC * _NS
_EPW = _E // _NW
_CH = 80
_NCHUNK = _EPW // _CH
_CHD = 80
_NCHD = _EPW // _CHD
_RPT = 640
_NPAD = _NS * _RPT
_BR = 1000

_sc_mesh = plsc.VectorSubcoreMesh(core_axis_name="c", subcore_axis_name="s")


@functools.partial(
    pl.kernel,
    out_type=jax.ShapeDtypeStruct((_NC, _NPAD), jnp.float32),
    mesh=_sc_mesh,
    scratch_types=[
        pltpu.VMEM((_NCHD, _CHD), jnp.int32),
        pltpu.VMEM((_CHD,), jnp.float32),
        pltpu.VMEM((_RPT,), jnp.float32),
        pltpu.VMEM_SHARED((_NPAD,), jnp.float32),
    ],
)
def _deg_kernel(dst_hbm, degp_hbm, dst_v, ones_v, zero_v, deg_sh):
    cid = lax.axis_index("c")
    sid = lax.axis_index("s")
    pltpu.sync_copy(dst_hbm.at[cid, sid], dst_v)
    for i in range(_CHD // 16):
        ones_v[pl.ds(i * 16, 16)] = jnp.full((16,), 1.0, jnp.float32)
    for i in range(_RPT // 16):
        zero_v[pl.ds(i * 16, 16)] = jnp.zeros((16,), jnp.float32)
    pltpu.sync_copy(zero_v, deg_sh.at[pl.ds(sid * _RPT, _RPT)])
    plsc.subcore_barrier()

    def body(k, carry):
        pltpu.sync_copy(ones_v, deg_sh.at[dst_v.at[k]], add=True)
        return carry

    lax.fori_loop(0, _NCHD, body, 0)
    plsc.subcore_barrier()
    pltpu.sync_copy(deg_sh.at[pl.ds(sid * _RPT, _RPT)],
                    degp_hbm.at[cid, pl.ds(sid * _RPT, _RPT)])


_NBUF = 2


@functools.partial(
    pl.kernel,
    out_type=jax.ShapeDtypeStruct((_NC, _NPAD, _D), jnp.float32),
    mesh=_sc_mesh,
    scratch_types=[
        pltpu.VMEM((_EPW,), jnp.int32),
        pltpu.VMEM((_NCHUNK, _CH), jnp.int32),
        [pltpu.VMEM((_CH, _D), jnp.float32) for _ in range(_NBUF)],
        pltpu.VMEM_SHARED((_NPAD, _D), jnp.float32),
        [pltpu.SemaphoreType.DMA for _ in range(_NBUF)],
    ],
)
def _agg_kernel(u_hbm, src_hbm, dst_hbm, aggp_hbm,
                src_v, dst_v, rows_v, agg_sh, sems):
    cid = lax.axis_index("c")
    sid = lax.axis_index("s")
    def zrow(r, c):
        for q in range(_D // 16):
            rows_v[0][r, pl.ds(q * 16, 16)] = jnp.zeros((16,), jnp.float32)
        return c

    lax.fori_loop(0, _CH, zrow, 0)
    for z in range(_RPT // _CH):
        pltpu.sync_copy(rows_v[0],
                        agg_sh.at[pl.ds(sid * _RPT + z * _CH, _CH)])
    pltpu.sync_copy(src_hbm.at[cid, sid], src_v)
    pltpu.sync_copy(dst_hbm.at[cid, sid], dst_v)
    plsc.subcore_barrier()

    def gather(k, b):
        pltpu.async_copy(u_hbm.at[cid].at[src_v.at[pl.ds(k * _CH, _CH)]],
                         rows_v[b], sems[b])

    def gather_wait(k, b):
        pltpu.make_async_copy(u_hbm.at[cid].at[src_v.at[pl.ds(k * _CH, _CH)]],
                              rows_v[b], sems[b]).wait()

    for b in range(_NBUF):
        gather(b, b)

    def group(g, carry):
        k0 = g * _NBUF
        for b in range(_NBUF):
            k = k0 + b
            gather_wait(k, b)
            pltpu.sync_copy(rows_v[b], agg_sh.at[dst_v.at[k]], add=True)

            @pl.when(k + _NBUF < _NCHUNK)
            def _():
                gather(k + _NBUF, b)
        return carry

    lax.fori_loop(0, _NCHUNK // _NBUF, group, 0)
    k_tail = (_NCHUNK // _NBUF) * _NBUF
    gather_wait(k_tail, 0)
    pltpu.sync_copy(rows_v[0], agg_sh.at[dst_v.at[k_tail]], add=True)
    plsc.subcore_barrier()
    pltpu.sync_copy(agg_sh.at[pl.ds(sid * _RPT, _RPT)],
                    aggp_hbm.at[cid, pl.ds(sid * _RPT, _RPT)])


@functools.partial(
    pl.kernel,
    out_type=(jax.ShapeDtypeStruct((_NC, _NPAD, _D), jnp.float32),
              jax.ShapeDtypeStruct((_NC, _NPAD, _D), jnp.float32)),
    mesh=_sc_mesh,
    scratch_types=[
        pltpu.VMEM((_EPW,), jnp.int32),
        pltpu.VMEM((_NCHUNK, _CH), jnp.int32),
        [pltpu.VMEM((_CH, _D), jnp.float32) for _ in range(_NBUF)],
        pltpu.VMEM_SHARED((_NPAD, _D), jnp.float32),
        [pltpu.SemaphoreType.DMA for _ in range(_NBUF)],
    ],
)
def _step_kernel(p_hbm, u_hbm, d2_hbm, c_hbm, src_hbm, dst_hbm,
                 pout_hbm, uout_hbm, src_v, dst_v, rows_v, agg_sh, sems):
    cid = lax.axis_index("c")
    sid = lax.axis_index("s")

    def zrow(r, c):
        for q in range(_D // 16):
            rows_v[0][r, pl.ds(q * 16, 16)] = jnp.zeros((16,), jnp.float32)
        return c

    lax.fori_loop(0, _CH, zrow, 0)
    for z in range(_RPT // _CH):
        pltpu.sync_copy(rows_v[0],
                        agg_sh.at[pl.ds(sid * _RPT + z * _CH, _CH)])

    def vpass(mul):
        def body(r, c):
            for q in range(_D // 16):
                sl = pl.ds(q * 16, 16)
                if mul:
                    rows_v[0][r, sl] = rows_v[0][r, sl] * rows_v[1][r, sl]
                else:
                    rows_v[0][r, sl] = rows_v[0][r, sl] + rows_v[1][r, sl]
            return c

        lax.fori_loop(0, _CH, body, 0)

    for zc in range(_RPT // _CH):
        rr = pl.ds(sid * _RPT + zc * _CH, _CH)
        pltpu.sync_copy(p_hbm.at[0, rr], rows_v[0])
        pltpu.sync_copy(p_hbm.at[1, rr], rows_v[1])
        vpass(False)
        pltpu.sync_copy(u_hbm.at[cid, rr], rows_v[1])
        vpass(False)
        pltpu.sync_copy(d2_hbm.at[rr], rows_v[1])
        vpass(True)
        pltpu.sync_copy(c_hbm.at[rr], rows_v[1])
        vpass(False)
        pltpu.sync_copy(rows_v[0], uout_hbm.at[cid, rr])
    plsc.subcore_barrier()

    pltpu.sync_copy(src_hbm.at[cid, sid], src_v)
    pltpu.sync_copy(dst_hbm.at[cid, sid], dst_v)

    def gather(k, b):
        pltpu.async_copy(uout_hbm.at[cid].at[src_v.at[pl.ds(k * _CH, _CH)]],
                         rows_v[b], sems[b])

    def gather_wait(k, b):
        pltpu.make_async_copy(
            uout_hbm.at[cid].at[src_v.at[pl.ds(k * _CH, _CH)]],
            rows_v[b], sems[b]).wait()

    for b in range(_NBUF):
        gather(b, b)

    def group(g, carry):
        k0 = g * _NBUF
        for b in range(_NBUF):
            k = k0 + b
            gather_wait(k, b)
            pltpu.sync_copy(rows_v[b], agg_sh.at[dst_v.at[k]], add=True)

            @pl.when(k + _NBUF < _NCHUNK)
            def _():
                gather(k + _NBUF, b)
        return carry

    lax.fori_loop(0, _NCHUNK // _NBUF, group, 0)
    k_tail = (_NCHUNK // _NBUF) * _NBUF
    gather_wait(k_tail, 0)
    pltpu.sync_copy(rows_v[0], agg_sh.at[dst_v.at[k_tail]], add=True)
    plsc.subcore_barrier()
    pltpu.sync_copy(agg_sh.at[pl.ds(sid * _RPT, _RPT)],
                    pout_hbm.at[cid, pl.ds(sid * _RPT, _RPT)])


def _dense1_body(x_ref, w_ref, b_ref, o_ref):
    o_ref[...] = jnp.dot(x_ref[...], w_ref[...],
                         preferred_element_type=jnp.float32) + b_ref[...]


def _dinv_body(degp_ref, o_ref):
    deg = degp_ref[0:1, :] + degp_ref[1:2, :] + 1.0
    o_ref[...] = lax.rsqrt(deg)


_BRP = 1280


def _prep_body(dinv_ref, h_ref, d2_ref, c_ref, u_ref):
    d = dinv_ref[...]
    d2_ref[...] = jnp.broadcast_to(0.5 * d * d, (_BRP, _D))
    c = (0.5 * d) * h_ref[...]
    c_ref[...] = c
    u_ref[0] = 2.0 * c
    u_ref[1] = 2.0 * c


def _combine_body(aggp_ref, u_ref, h_ref, dinv_ref, o_ref):
    dinv = dinv_ref[...]
    s = aggp_ref[0] + aggp_ref[1] + u_ref[0]
    u = 0.5 * dinv * (dinv * s + h_ref[...])
    o_ref[0] = u
    o_ref[1] = u


def _final_body(aggp_ref, u_ref, h_ref, dinv_ref, w_ref, b_ref, o_ref):
    s = aggp_ref[0] + aggp_ref[1] + u_ref[0]
    z = 0.5 * (dinv_ref[...] * s + h_ref[...])
    o_ref[...] = jnp.dot(jax.nn.relu(z), w_ref[...],
                         preferred_element_type=jnp.float32) + b_ref[...]


def _row_specs(extras):
    specs = [
        pl.BlockSpec((_NC, _BR, _D), lambda i: (0, i, 0)),
        pl.BlockSpec((_NC, _BR, _D), lambda i: (0, i, 0)),
        pl.BlockSpec((_BR, _D), lambda i: (i, 0)),
        pl.BlockSpec((_BR, 1), lambda i: (i, 0)),
    ]
    specs += [pl.BlockSpec(s, lambda i: (0, 0)) for s in extras]
    return specs


def kernel(x, edge_index, W1, b1, W2, b2):
    src = edge_index[0].astype(jnp.int32).reshape(_NC, _NS, _EPW)
    dst = edge_index[1].astype(jnp.int32).reshape(_NC, _NS, _NCHUNK, _CH)

    h0 = pl.pallas_call(
        _dense1_body,
        grid=(_N // _BR,),
        in_specs=[
            pl.BlockSpec((_BR, _D), lambda i: (i, 0)),
            pl.BlockSpec((_D, _D), lambda i: (0, 0)),
            pl.BlockSpec((1, _D), lambda i: (0, 0)),
        ],
        out_specs=pl.BlockSpec((_BR, _D), lambda i: (i, 0)),
        out_shape=jax.ShapeDtypeStruct((_N, _D), jnp.float32),
    )(x, W1, b1.reshape(1, _D))

    degp = _deg_kernel(dst.reshape(_NC, _NS, _NCHD, _CHD))

    dinv_row = pl.pallas_call(
        _dinv_body,
        out_shape=jax.ShapeDtypeStruct((1, _NPAD), jnp.float32),
    )(degp)
    dinv_col = dinv_row.reshape(_NPAD, 1)

    h0p = jnp.concatenate(
        [h0, jnp.zeros((_NPAD - _N, _D), jnp.float32)], axis=0)
    d2f, cf, u = pl.pallas_call(
        _prep_body,
        grid=(_NPAD // _BRP,),
        in_specs=[
            pl.BlockSpec((_BRP, 1), lambda i: (i, 0)),
            pl.BlockSpec((_BRP, _D), lambda i: (i, 0)),
        ],
        out_specs=[
            pl.BlockSpec((_BRP, _D), lambda i: (i, 0)),
            pl.BlockSpec((_BRP, _D), lambda i: (i, 0)),
            pl.BlockSpec((_NC, _BRP, _D), lambda i: (0, i, 0)),
        ],
        out_shape=[
            jax.ShapeDtypeStruct((_NPAD, _D), jnp.float32),
            jax.ShapeDtypeStruct((_NPAD, _D), jnp.float32),
            jax.ShapeDtypeStruct((_NC, _NPAD, _D), jnp.float32),
        ],
    )(dinv_col, h0p)

    aggp = _agg_kernel(u, src, dst)
    for _ in range(_K - 1):
        aggp, u = _step_kernel(aggp, u, d2f, cf, src, dst)
    out = pl.pallas_call(
        _final_body,
        grid=(_N // _BR,),
        in_specs=_row_specs([(_D, _D), (1, _D)]),
        out_specs=pl.BlockSpec((_BR, _D), lambda i: (i, 0)),
        out_shape=jax.ShapeDtypeStruct((_N, _D), jnp.float32),
    )(aggp, u, h0, dinv_col, W2, b2.reshape(1, _D))
    return out


# speedup vs baseline: 1.3589x; 1.3589x over previous
import functools

import jax
import jax.numpy as jnp
from jax import lax
from jax.experimental import pallas as pl
from jax.experimental.pallas import tpu as pltpu
from jax.experimental.pallas import tpu_sc as plsc

_N = 10000
_E = 320000
_D = 128
_K = 5

_NC = 2
_NS = 16
_NW = _NC * _NS
_EPW = _E // _NW
_CH = 88
_EPWP = 10032
_NCHUNK = _EPWP // _CH
_CHD = 80
_NCHD = _EPW // _CHD
_RPT = 640
_NPAD = _NS * _RPT
_BR = 1000

_sc_mesh = plsc.VectorSubcoreMesh(core_axis_name="c", subcore_axis_name="s")


@functools.partial(
    pl.kernel,
    out_type=jax.ShapeDtypeStruct((_NC, _NPAD), jnp.float32),
    mesh=_sc_mesh,
    scratch_types=[
        pltpu.VMEM((_NCHD, _CHD), jnp.int32),
        pltpu.VMEM((_CHD,), jnp.float32),
        pltpu.VMEM((_RPT,), jnp.float32),
        pltpu.VMEM_SHARED((_NPAD,), jnp.float32),
    ],
)
def _deg_kernel(dst_hbm, degp_hbm, dst_v, ones_v, zero_v, deg_sh):
    cid = lax.axis_index("c")
    sid = lax.axis_index("s")
    pltpu.sync_copy(dst_hbm.at[cid, sid], dst_v)
    for i in range(_CHD // 16):
        ones_v[pl.ds(i * 16, 16)] = jnp.full((16,), 1.0, jnp.float32)
    for i in range(_RPT // 16):
        zero_v[pl.ds(i * 16, 16)] = jnp.zeros((16,), jnp.float32)
    pltpu.sync_copy(zero_v, deg_sh.at[pl.ds(sid * _RPT, _RPT)])
    plsc.subcore_barrier()

    def body(k, carry):
        pltpu.sync_copy(ones_v, deg_sh.at[dst_v.at[k]], add=True)
        return carry

    lax.fori_loop(0, _NCHD, body, 0)
    plsc.subcore_barrier()
    pltpu.sync_copy(deg_sh.at[pl.ds(sid * _RPT, _RPT)],
                    degp_hbm.at[cid, pl.ds(sid * _RPT, _RPT)])


_NBUF = 2


@functools.partial(
    pl.kernel,
    out_type=jax.ShapeDtypeStruct((_NC, _NPAD, _D), jnp.float32),
    mesh=_sc_mesh,
    scratch_types=[
        pltpu.VMEM((_EPWP,), jnp.int32),
        pltpu.VMEM((_NCHUNK, _CH), jnp.int32),
        [pltpu.VMEM((_CH, _D), jnp.float32) for _ in range(_NBUF)],
        pltpu.VMEM_SHARED((_NPAD, _D), jnp.float32),
        [pltpu.SemaphoreType.DMA for _ in range(_NBUF)],
    ],
)
def _agg_kernel(u_hbm, src_hbm, dst_hbm, aggp_hbm,
                src_v, dst_v, rows_v, agg_sh, sems):
    cid = lax.axis_index("c")
    sid = lax.axis_index("s")
    def zrow(r, c):
        for q in range(_D // 16):
            rows_v[0][r, pl.ds(q * 16, 16)] = jnp.zeros((16,), jnp.float32)
        return c

    lax.fori_loop(0, _CH, zrow, 0)
    for z in range(_RPT // 80):
        pltpu.sync_copy(rows_v[0].at[pl.ds(0, 80)],
                        agg_sh.at[pl.ds(sid * _RPT + z * 80, 80)])
    pltpu.sync_copy(src_hbm.at[cid, sid], src_v)
    pltpu.sync_copy(dst_hbm.at[cid, sid], dst_v)
    plsc.subcore_barrier()

    def gather(k, b):
        pltpu.async_copy(u_hbm.at[cid].at[src_v.at[pl.ds(k * _CH, _CH)]],
                         rows_v[b], sems[b])

    def gather_wait(k, b):
        pltpu.make_async_copy(u_hbm.at[cid].at[src_v.at[pl.ds(k * _CH, _CH)]],
                              rows_v[b], sems[b]).wait()

    for b in range(_NBUF):
        gather(b, b)

    def group(g, carry):
        k0 = g * _NBUF
        for b in range(_NBUF):
            k = k0 + b
            gather_wait(k, b)
            pltpu.sync_copy(rows_v[b], agg_sh.at[dst_v.at[k]], add=True)

            @pl.when(k + _NBUF < _NCHUNK)
            def _():
                gather(k + _NBUF, b)
        return carry

    lax.fori_loop(0, _NCHUNK // _NBUF, group, 0)
    plsc.subcore_barrier()
    pltpu.sync_copy(agg_sh.at[pl.ds(sid * _RPT, _RPT)],
                    aggp_hbm.at[cid, pl.ds(sid * _RPT, _RPT)])


def _dense1_body(x_ref, w_ref, b_ref, o_ref):
    o_ref[...] = jnp.dot(x_ref[...], w_ref[...],
                         preferred_element_type=jnp.float32) + b_ref[...]


def _dinv_body(degp_ref, o_ref):
    deg = degp_ref[0:1, :] + degp_ref[1:2, :] + 1.0
    o_ref[...] = lax.rsqrt(deg)


def _u0_body(dinv_ref, h_ref, o_ref):
    u = dinv_ref[...] * h_ref[...]
    o_ref[0] = u
    o_ref[1] = u


def _combine_body(aggp_ref, u_ref, h_ref, dinv_ref, o_ref):
    dinv = dinv_ref[...]
    s = aggp_ref[0] + aggp_ref[1] + u_ref[0]
    u = 0.5 * dinv * (dinv * s + h_ref[...])
    o_ref[0] = u
    o_ref[1] = u


def _final_body(aggp_ref, u_ref, h_ref, dinv_ref, w_ref, b_ref, o_ref):
    s = aggp_ref[0] + aggp_ref[1] + u_ref[0]
    z = 0.5 * (dinv_ref[...] * s + h_ref[...])
    o_ref[...] = jnp.dot(jax.nn.relu(z), w_ref[...],
                         preferred_element_type=jnp.float32) + b_ref[...]


def _row_specs(extras):
    specs = [
        pl.BlockSpec((_NC, _BR, _D), lambda i: (0, i, 0)),
        pl.BlockSpec((_NC, _BR, _D), lambda i: (0, i, 0)),
        pl.BlockSpec((_BR, _D), lambda i: (i, 0)),
        pl.BlockSpec((_BR, 1), lambda i: (i, 0)),
    ]
    specs += [pl.BlockSpec(s, lambda i: (0, 0)) for s in extras]
    return specs


def kernel(x, edge_index, W1, b1, W2, b2):
    src0 = edge_index[0].astype(jnp.int32).reshape(_NC * _NS, _EPW)
    dst0 = edge_index[1].astype(jnp.int32).reshape(_NC * _NS, _EPW)
    padn = _EPWP - _EPW
    wids = jnp.arange(_NC * _NS, dtype=jnp.int32)[:, None]
    pads = jnp.arange(padn, dtype=jnp.int32)[None, :]
    pad_src = (pads * 17 + wids * 311) % _N
    pad_dst = _N + (pads + wids * 7) % (_NPAD - _N)
    src = jnp.concatenate([src0, pad_src], axis=1).reshape(_NC, _NS, _EPWP)
    dst = jnp.concatenate([dst0, pad_dst], axis=1).reshape(
        _NC, _NS, _NCHUNK, _CH)

    h0 = pl.pallas_call(
        _dense1_body,
        grid=(_N // _BR,),
        in_specs=[
            pl.BlockSpec((_BR, _D), lambda i: (i, 0)),
            pl.BlockSpec((_D, _D), lambda i: (0, 0)),
            pl.BlockSpec((1, _D), lambda i: (0, 0)),
        ],
        out_specs=pl.BlockSpec((_BR, _D), lambda i: (i, 0)),
        out_shape=jax.ShapeDtypeStruct((_N, _D), jnp.float32),
    )(x, W1, b1.reshape(1, _D))

    degp = _deg_kernel(dst0.reshape(_NC, _NS, _NCHD, _CHD))

    dinv_row = pl.pallas_call(
        _dinv_body,
        out_shape=jax.ShapeDtypeStruct((1, _NPAD), jnp.float32),
    )(degp)
    dinv_col = dinv_row.reshape(_NPAD, 1)[:_N]

    u = pl.pallas_call(
        _u0_body,
        grid=(_N // _BR,),
        in_specs=[
            pl.BlockSpec((_BR, 1), lambda i: (i, 0)),
            pl.BlockSpec((_BR, _D), lambda i: (i, 0)),
        ],
        out_specs=pl.BlockSpec((_NC, _BR, _D), lambda i: (0, i, 0)),
        out_shape=jax.ShapeDtypeStruct((_NC, _N, _D), jnp.float32),
    )(dinv_col, h0)

    for _ in range(_K - 1):
        aggp = _agg_kernel(u, src, dst)
        u = pl.pallas_call(
            _combine_body,
            grid=(_N // _BR,),
            in_specs=_row_specs([]),
            out_specs=pl.BlockSpec((_NC, _BR, _D), lambda i: (0, i, 0)),
            out_shape=jax.ShapeDtypeStruct((_NC, _N, _D), jnp.float32),
        )(aggp, u, h0, dinv_col)

    aggp = _agg_kernel(u, src, dst)
    out = pl.pallas_call(
        _final_body,
        grid=(_N // _BR,),
        in_specs=_row_specs([(_D, _D), (1, _D)]),
        out_specs=pl.BlockSpec((_BR, _D), lambda i: (i, 0)),
        out_shape=jax.ShapeDtypeStruct((_N, _D), jnp.float32),
    )(aggp, u, h0, dinv_col, W2, b2.reshape(1, _D))
    return out
